# baseline (speedup 1.0000x reference)
import functools

import jax
import jax.numpy as jnp
from jax import lax
from jax.experimental import pallas as pl
from jax.experimental.pallas import tpu as pltpu, tpu_sc as plsc

OUT_H = 7
OUT_W = 7
SPATIAL_SCALE = 0.25
SAMPLING = 2

NUM_WORKERS = 32
CELLS_PER_CHUNK = 8
CONTRIB = 16
ROWS_PER_CHUNK = CELLS_PER_CHUNK * CONTRIB
LANES = 16


def _axis_terms(start, bin_sz, extent, p, g, d):
    s = start + (p + (g + 0.5) * (1.0 / SAMPLING)) * bin_sz
    valid = (s >= -1.0) & (s <= extent)
    sc = jnp.clip(s, 0.0, extent - 1.0)
    i0 = jnp.clip(jnp.floor(sc).astype(jnp.int32), 0, extent - 1)
    frac = sc - i0.astype(jnp.float32)
    idx = jnp.minimum(i0 + d, extent - 1)
    w = jnp.where(d == 0, 1.0 - frac, frac)
    return idx, jnp.where(valid, w, 0.0)


def _build_indices_weights(rois, N, H, W):
    R = rois.shape[0]
    T = OUT_H * OUT_W * CONTRIB
    bidx = rois[:, 0:1].astype(jnp.int32)
    sx = rois[:, 1:2] * SPATIAL_SCALE - 0.5
    sy = rois[:, 2:3] * SPATIAL_SCALE - 0.5
    ex = rois[:, 3:4] * SPATIAL_SCALE - 0.5
    ey = rois[:, 4:5] * SPATIAL_SCALE - 0.5
    bw = (ex - sx) * (1.0 / OUT_W)
    bh = (ey - sy) * (1.0 / OUT_H)
    t = lax.broadcasted_iota(jnp.int32, (R, T), 1)
    ph = t // 112
    pw = (t // 16) % 7
    gy = (t // 8) % 2
    dy = (t // 4) % 2
    gx = (t // 2) % 2
    dx = t % 2
    yi, yw = _axis_terms(sy, bh, H, ph.astype(jnp.float32),
                         gy.astype(jnp.float32), dy)
    xi, xw = _axis_terms(sx, bw, W, pw.astype(jnp.float32),
                         gx.astype(jnp.float32), dx)
    idx = bidx * (H * W) + yi * W + xi
    wgt = yw * xw * (1.0 / (SAMPLING * SAMPLING))
    return idx.reshape(-1), wgt.astype(jnp.float32).reshape(-1)


def _sc_pool(table, idx, wgt, num_cells, C):
    chunks_total = num_cells // CELLS_PER_CHUNK
    chunks_per_worker = chunks_total // NUM_WORKERS
    mesh = plsc.VectorSubcoreMesh(core_axis_name="c", subcore_axis_name="s")
    rows_per_worker = chunks_per_worker * ROWS_PER_CHUNK

    @functools.partial(
        pl.kernel,
        out_type=jax.ShapeDtypeStruct((num_cells, C), jnp.float32),
        mesh=mesh,
        scratch_types=[
            pltpu.VMEM((chunks_per_worker * ROWS_PER_CHUNK,), jnp.int32),
            pltpu.VMEM((chunks_per_worker * ROWS_PER_CHUNK,), jnp.float32),
            pltpu.VMEM((ROWS_PER_CHUNK, C), jnp.float32),
            pltpu.VMEM((ROWS_PER_CHUNK, C), jnp.float32),
            pltpu.VMEM((CELLS_PER_CHUNK, C), jnp.float32),
            pltpu.VMEM((CELLS_PER_CHUNK, C), jnp.float32),
            pltpu.SemaphoreType.DMA,
            pltpu.SemaphoreType.DMA,
            pltpu.SemaphoreType.DMA,
            pltpu.SemaphoreType.DMA,
        ],
    )
    def run(table_hbm, idx_hbm, wgt_hbm, out_hbm,
            idx_all, wgt_all, rows_v0, rows_v1, out_v0, out_v1,
            gsem0, gsem1, osem0, osem1):
        wid = lax.axis_index("s") * 2 + lax.axis_index("c")
        chunk0 = wid * chunks_per_worker
        row0 = chunk0 * ROWS_PER_CHUNK
        pltpu.sync_copy(idx_hbm.at[pl.ds(row0, rows_per_worker)], idx_all)
        pltpu.sync_copy(wgt_hbm.at[pl.ds(row0, rows_per_worker)], wgt_all)
        slots = ((rows_v0, out_v0, gsem0, osem0),
                 (rows_v1, out_v1, gsem1, osem1))

        def out_copy(g, slot):
            _, out_v, _, osem = slot
            cell_off = (chunk0 + g) * CELLS_PER_CHUNK
            return pltpu.make_async_copy(
                out_v, out_hbm.at[pl.ds(cell_off, CELLS_PER_CHUNK)], osem)

        def start(g, slot):
            rows_v, _, gsem, _ = slot
            pltpu.make_async_copy(table_hbm.at[idx_all.at[pl.ds(g * ROWS_PER_CHUNK, ROWS_PER_CHUNK)]], rows_v, gsem).start()

        def finish(g, slot):
            rows_v, out_v, gsem, _ = slot
            pltpu.make_async_copy(table_hbm.at[idx_all.at[pl.ds(g * ROWS_PER_CHUNK, ROWS_PER_CHUNK)]], rows_v, gsem).wait()

            @pl.when(g >= 2)
            def _():
                out_copy(g - 2, slot).wait()

            def cell_body(c, carry):
                cb = c * CONTRIB
                wv = wgt_all[pl.ds(g * ROWS_PER_CHUNK + cb, CONTRIB)]
                accs = [jnp.zeros((LANES,), jnp.float32)] * (C // LANES)
                dnums = lax.GatherDimensionNumbers(
                    offset_dims=(), collapsed_slice_dims=(0,),
                    start_index_map=(0,))
                for k in range(CONTRIB):
                    w = lax.gather(
                        wv, jnp.full((LANES, 1), k, dtype=jnp.int32), dnums,
                        slice_sizes=(1,),
                        mode=lax.GatherScatterMode.PROMISE_IN_BOUNDS)
                    for v in range(C // LANES):
                        accs[v] = accs[v] + w * rows_v[cb + k, pl.ds(v * LANES, LANES)]
                for v in range(C // LANES):
                    out_v[c, pl.ds(v * LANES, LANES)] = accs[v]
                return carry

            for c in range(CELLS_PER_CHUNK):
                cell_body(c, 0)
            out_copy(g, slot).start()

        start(0, slots[0])

        def outer(gp, carry):
            g = gp * 2

            @pl.when(g + 1 < chunks_per_worker)
            def _():
                start(g + 1, slots[1])

            finish(g, slots[0])

            @pl.when(g + 2 < chunks_per_worker)
            def _():
                start(g + 2, slots[0])

            @pl.when(g + 1 < chunks_per_worker)
            def _():
                finish(g + 1, slots[1])

            return carry

        lax.fori_loop(0, (chunks_per_worker + 1) // 2, outer, 0)
        out_copy(chunks_per_worker - 2, slots[0]).wait()
        out_copy(chunks_per_worker - 1, slots[1]).wait()

    return run(table, idx, wgt)


def kernel(input, rois):
    N, C, H, W = input.shape
    R = rois.shape[0]
    table = jnp.transpose(input, (0, 2, 3, 1)).reshape(N * H * W, C)
    idx, wgt = _build_indices_weights(rois, N, H, W)
    num_cells = R * OUT_H * OUT_W
    out = _sc_pool(table, idx, wgt, num_cells, C)
    return out.reshape(R, OUT_H * OUT_W, C).transpose(0, 2, 1).reshape(R, C, OUT_H, OUT_W)


# speedup vs baseline: 1.6994x; 1.6994x over previous
import functools

import jax
import jax.numpy as jnp
from jax import lax
from jax.experimental import pallas as pl
from jax.experimental.pallas import tpu as pltpu, tpu_sc as plsc

OUT_H = 7
OUT_W = 7
SPATIAL_SCALE = 0.25
SAMPLING = 2

NUM_WORKERS = 32
CELLS_PER_CHUNK = 8
CONTRIB = 16
ROWS_PER_CHUNK = CELLS_PER_CHUNK * CONTRIB
LANES = 16


def _axis_terms(start, bin_sz, extent, p, g, d):
    s = start + (p + (g + 0.5) * (1.0 / SAMPLING)) * bin_sz
    valid = (s >= -1.0) & (s <= extent)
    sc = jnp.clip(s, 0.0, extent - 1.0)
    i0 = jnp.clip(jnp.floor(sc).astype(jnp.int32), 0, extent - 1)
    frac = sc - i0.astype(jnp.float32)
    idx = jnp.minimum(i0 + d, extent - 1)
    w = jnp.where(d == 0, 1.0 - frac, frac)
    return idx, jnp.where(valid, w, 0.0)


def _build_indices_weights(rois, N, H, W):
    R = rois.shape[0]
    T = OUT_H * OUT_W * CONTRIB
    bidx = rois[:, 0:1].astype(jnp.int32)
    sx = rois[:, 1:2] * SPATIAL_SCALE - 0.5
    sy = rois[:, 2:3] * SPATIAL_SCALE - 0.5
    ex = rois[:, 3:4] * SPATIAL_SCALE - 0.5
    ey = rois[:, 4:5] * SPATIAL_SCALE - 0.5
    bw = (ex - sx) * (1.0 / OUT_W)
    bh = (ey - sy) * (1.0 / OUT_H)
    t = lax.broadcasted_iota(jnp.int32, (R, T), 1)
    ph = t // 112
    pw = (t // 16) % 7
    gy = (t // 8) % 2
    dy = (t // 4) % 2
    gx = (t // 2) % 2
    dx = t % 2
    yi, yw = _axis_terms(sy, bh, H, ph.astype(jnp.float32),
                         gy.astype(jnp.float32), dy)
    xi, xw = _axis_terms(sx, bw, W, pw.astype(jnp.float32),
                         gx.astype(jnp.float32), dx)
    idx = bidx * (H * W) + yi * W + xi
    wgt = yw * xw * (1.0 / (SAMPLING * SAMPLING))
    return idx.reshape(-1), wgt.astype(jnp.float32).reshape(-1)


def _sc_pool(table, idx, wgt, num_cells, C):
    chunks_total = num_cells // CELLS_PER_CHUNK
    chunks_per_worker = chunks_total // NUM_WORKERS
    mesh = plsc.VectorSubcoreMesh(core_axis_name="c", subcore_axis_name="s")
    rows_per_worker = chunks_per_worker * ROWS_PER_CHUNK

    @functools.partial(
        pl.kernel,
        out_type=jax.ShapeDtypeStruct((num_cells, C), jnp.float32),
        mesh=mesh,
        scratch_types=[
            pltpu.VMEM((chunks_per_worker * ROWS_PER_CHUNK,), jnp.int32),
            pltpu.VMEM((chunks_per_worker * ROWS_PER_CHUNK,), jnp.float32),
            pltpu.VMEM((ROWS_PER_CHUNK, C), jnp.float32),
            pltpu.VMEM((ROWS_PER_CHUNK, C), jnp.float32),
            pltpu.VMEM((CELLS_PER_CHUNK, C), jnp.float32),
            pltpu.VMEM((CELLS_PER_CHUNK, C), jnp.float32),
            pltpu.SemaphoreType.DMA,
            pltpu.SemaphoreType.DMA,
            pltpu.SemaphoreType.DMA,
            pltpu.SemaphoreType.DMA,
        ],
    )
    def run(table_hbm, idx_hbm, wgt_hbm, out_hbm,
            idx_all, wgt_all, rows_v0, rows_v1, out_v0, out_v1,
            gsem0, gsem1, osem0, osem1):
        wid = lax.axis_index("s") * 2 + lax.axis_index("c")
        chunk0 = wid * chunks_per_worker
        row0 = chunk0 * ROWS_PER_CHUNK
        pltpu.sync_copy(idx_hbm.at[pl.ds(row0, rows_per_worker)], idx_all)
        pltpu.sync_copy(wgt_hbm.at[pl.ds(row0, rows_per_worker)], wgt_all)
        slots = ((rows_v0, out_v0, gsem0, osem0),
                 (rows_v1, out_v1, gsem1, osem1))

        def out_copy(g, slot):
            _, out_v, _, osem = slot
            cell_off = (chunk0 + g) * CELLS_PER_CHUNK
            return pltpu.make_async_copy(
                out_v, out_hbm.at[pl.ds(cell_off, CELLS_PER_CHUNK)], osem)

        def start(g, slot):
            rows_v, _, gsem, _ = slot
            pltpu.make_async_copy(table_hbm.at[idx_all.at[pl.ds(g * ROWS_PER_CHUNK, ROWS_PER_CHUNK)]], rows_v, gsem).start()

        def finish(g, slot):
            rows_v, out_v, gsem, _ = slot
            pltpu.make_async_copy(table_hbm.at[idx_all.at[pl.ds(g * ROWS_PER_CHUNK, ROWS_PER_CHUNK)]], rows_v, gsem).wait()

            @pl.when(g >= 2)
            def _():
                out_copy(g - 2, slot).wait()

            def cell_body(c, carry):
                cb = c * CONTRIB
                wv = wgt_all[pl.ds(g * ROWS_PER_CHUNK + cb, CONTRIB)]
                accs = [jnp.zeros((LANES,), jnp.float32)] * (C // LANES)
                dnums = lax.GatherDimensionNumbers(
                    offset_dims=(), collapsed_slice_dims=(0,),
                    start_index_map=(0,))
                for k in range(CONTRIB):
                    w = lax.gather(
                        wv, jnp.full((LANES, 1), k, dtype=jnp.int32), dnums,
                        slice_sizes=(1,),
                        mode=lax.GatherScatterMode.PROMISE_IN_BOUNDS)
                    for v in range(C // LANES):
                        accs[v] = accs[v] + w * rows_v[cb + k, pl.ds(v * LANES, LANES)]
                for v in range(C // LANES):
                    out_v[c, pl.ds(v * LANES, LANES)] = accs[v]
                return carry

            lax.fori_loop(0, CELLS_PER_CHUNK, cell_body, 0)
            out_copy(g, slot).start()

        start(0, slots[0])

        def outer(gp, carry):
            g = gp * 2

            @pl.when(g + 1 < chunks_per_worker)
            def _():
                start(g + 1, slots[1])

            finish(g, slots[0])

            @pl.when(g + 2 < chunks_per_worker)
            def _():
                start(g + 2, slots[0])

            @pl.when(g + 1 < chunks_per_worker)
            def _():
                finish(g + 1, slots[1])

            return carry

        lax.fori_loop(0, (chunks_per_worker + 1) // 2, outer, 0)
        out_copy(chunks_per_worker - 2, slots[0]).wait()
        out_copy(chunks_per_worker - 1, slots[1]).wait()

    return run(table, idx, wgt)


def kernel(input, rois):
    N, C, H, W = input.shape
    R = rois.shape[0]
    table = jnp.transpose(input, (0, 2, 3, 1)).reshape(N * H * W, C)
    idx, wgt = _build_indices_weights(rois, N, H, W)
    num_cells = R * OUT_H * OUT_W
    out = _sc_pool(table, idx, wgt, num_cells, C)
    return out.reshape(R, OUT_H * OUT_W, C).transpose(0, 2, 1).reshape(R, C, OUT_H, OUT_W)
